# baseline (speedup 1.0000x reference)
import jax
import jax.numpy as jnp
from jax import lax
from jax.experimental import pallas as pl
from jax.experimental.pallas import tpu as pltpu
from jax.experimental.pallas import tpu_sc as plsc

VOCAB_SIZES = [99991, 100003, 100019, 100043, 100049, 100057, 100069, 100103]
H = len(VOCAB_SIZES)
D = 64
B, S = 4, 2048

_off = []
_acc = 0
for _v in VOCAB_SIZES:
    _off.append(_acc)
    _acc += _v

NC, NS, L = 2, 16, 16
NW = NC * NS
SW = S * B // NW
SC_CHUNK = 16
NCHUNK = SW // SC_CHUNK

NBUF = 4
DEPTH = 2


def _body(ids_hbm, table_hbm, off_hbm, out_hbm, stag_v, zs_v, bufs_v, *sems):
    gsems = sems[:NBUF]
    wsems = sems[NBUF:]
    wid = lax.axis_index("s") * NC + lax.axis_index("c")
    b = wid // (NW // B)
    s0 = (wid % (NW // B)) * SW

    pltpu.sync_copy(ids_hbm.at[b, pl.ds(s0, SW)], stag_v)

    iota = lax.iota(jnp.int32, L)
    zero = iota - iota

    def zfill(k, _):
        zs_v[pl.ds(k * L, L)] = zero
        return 0

    lax.fori_loop(0, SW // L, zfill, 0)

    pltpu.async_copy(off_hbm.at[zs_v], stag_v, gsems[0], add=True).wait()

    w = [None] * NCHUNK

    def fire(j):
        bi = j % NBUF

        def grow(r, _):
            pltpu.async_copy(
                table_hbm.at[stag_v.at[j * SC_CHUNK + r]],
                bufs_v.at[bi].at[r],
                gsems[bi],
            )
            return 0

        lax.fori_loop(0, SC_CHUNK, grow, 0)

    def retire(j):
        bi = j % NBUF
        pltpu.make_async_copy(
            out_hbm.at[b, pl.ds(0, SC_CHUNK)],
            bufs_v.at[bi],
            gsems[bi],
        ).wait()
        w[j] = pltpu.async_copy(
            bufs_v.at[bi],
            out_hbm.at[b, pl.ds(s0 + j * SC_CHUNK, SC_CHUNK)],
            wsems[bi],
        )

    for j in range(NCHUNK):
        if j >= NBUF:
            w[j - NBUF].wait()
        fire(j)
        if j >= DEPTH:
            retire(j - DEPTH)
    for j in range(NCHUNK - DEPTH, NCHUNK):
        retire(j)
    for j in range(NCHUNK - NBUF, NCHUNK):
        w[j].wait()


@jax.jit
def kernel(input_ids, table):
    offs = jnp.asarray([_off], dtype=jnp.int32)
    mesh = plsc.VectorSubcoreMesh(core_axis_name="c", subcore_axis_name="s")
    out = pl.kernel(
        _body,
        mesh=mesh,
        out_type=jax.ShapeDtypeStruct((B, S, H, D), jnp.float32),
        compiler_params=pltpu.CompilerParams(use_tc_tiling_on_sc=False),
        scratch_types=[
            pltpu.VMEM((SW, H), jnp.int32),
            pltpu.VMEM((SW,), jnp.int32),
            pltpu.VMEM((NBUF, SC_CHUNK, H, D), jnp.float32),
        ]
        + [pltpu.SemaphoreType.DMA] * (2 * NBUF),
    )(input_ids, table, offs)
    return out
